# Initial kernel scaffold; baseline (speedup 1.0000x reference)
import jax
import jax.numpy as jnp
from jax.experimental import pallas as pl


def kernel(hidden_states, router_w, gate_w, up_w, down_w):
    raise NotImplementedError("write your pallas kernel here")


# speedup vs baseline: 1.3271x; 1.3271x over previous
import functools

import jax
import jax.numpy as jnp
from jax.experimental import pallas as pl
from jax.experimental.pallas import tpu as pltpu

_NE = 8
_D = 1024
_F = 2816
_S = 2048
_TB = 512


def _dotT(a, b):
    return jax.lax.dot_general(
        a, b, (((1,), (1,)), ((), ())), preferred_element_type=jnp.float32)


def _moe_body(x_ref, rw_ref, gw_ref, uw_ref, dw_ref, out_ref, wts_ref):
    e = pl.program_id(1)

    @pl.when(e == 0)
    def _router():
        xf = x_ref[...]
        logits = _dotT(xf, rw_ref[...])
        iota = jax.lax.broadcasted_iota(jnp.int32, logits.shape, 1)
        v1 = jnp.max(logits, axis=1, keepdims=True)
        i1 = jnp.min(jnp.where(logits == v1, iota, _NE), axis=1, keepdims=True)
        sel1 = iota == i1
        l2 = jnp.where(sel1, -jnp.inf, logits)
        v2 = jnp.max(l2, axis=1, keepdims=True)
        i2 = jnp.min(jnp.where(l2 == v2, iota, _NE), axis=1, keepdims=True)
        sel2 = iota == i2
        a = jnp.exp(v2 - v1)
        w1 = 1.0 / (1.0 + a)
        wts_ref[...] = jnp.where(sel1, w1, 0.0) + jnp.where(sel2, w1 * a, 0.0)
        out_ref[...] = jnp.zeros_like(out_ref)

    xb = x_ref[...].astype(jnp.bfloat16)
    g = _dotT(xb, gw_ref[0])
    u = _dotT(xb, uw_ref[0])
    h = (g * jax.nn.sigmoid(g) * u).astype(jnp.bfloat16)
    eo = _dotT(h, dw_ref[0])
    iota = jax.lax.broadcasted_iota(jnp.int32, (_TB, _NE), 1)
    w_e = jnp.sum(jnp.where(iota == e, wts_ref[...], 0.0), axis=1,
                  keepdims=True)
    out_ref[...] += w_e * eo


@jax.jit
def _moe(x, router_w, gw, uw, dw):
    nt = _S // _TB
    return pl.pallas_call(
        _moe_body,
        grid=(nt, _NE),
        in_specs=[
            pl.BlockSpec((_TB, _D), lambda t, e: (t, 0)),
            pl.BlockSpec((_NE, _D), lambda t, e: (0, 0)),
            pl.BlockSpec((1, _F, _D), lambda t, e: (e, 0, 0)),
            pl.BlockSpec((1, _F, _D), lambda t, e: (e, 0, 0)),
            pl.BlockSpec((1, _D, _F), lambda t, e: (e, 0, 0)),
        ],
        out_specs=pl.BlockSpec((_TB, _D), lambda t, e: (t, 0)),
        out_shape=jax.ShapeDtypeStruct((_S, _D), jnp.float32),
        scratch_shapes=[pltpu.VMEM((_TB, _NE), jnp.float32)],
        compiler_params=pltpu.CompilerParams(
            dimension_semantics=("arbitrary", "arbitrary"),
        ),
    )(x, router_w, gw, uw, dw)


def kernel(hidden_states, router_w, gate_w, up_w, down_w):
    B, S, D = hidden_states.shape
    x = hidden_states.reshape(S, D)
    out = _moe(x, router_w,
               gate_w.astype(jnp.bfloat16),
               up_w.astype(jnp.bfloat16),
               down_w.astype(jnp.bfloat16))
    return out.reshape(B, S, D)


# speedup vs baseline: 1.9274x; 1.4523x over previous
import functools

import jax
import jax.numpy as jnp
from jax import lax
from jax.experimental import pallas as pl
from jax.experimental.pallas import tpu as pltpu
from jax.experimental.pallas import tpu_sc as plsc

_NE = 8
_D = 1024
_F = 2816
_S = 2048
_K = 2
_BLK = 256
_NB = (_K * _S) // _BLK + (_NE - 1)
_P = _NB * _BLK
_NEG = -1e30


def _route_body(x_ref, rw_ref, dest_ref, wts_ref, be_ref, bv_ref):
    xf = x_ref[...]
    rw = rw_ref[...]
    logits = jax.lax.dot_general(
        rw, xf, (((1,), (1,)), ((), ())),
        preferred_element_type=jnp.float32)
    iota_e = jax.lax.broadcasted_iota(jnp.int32, (_NE, _S), 0)
    v1 = jnp.max(logits, axis=0, keepdims=True)
    i1 = jnp.min(jnp.where(logits == v1, iota_e, _NE), axis=0, keepdims=True)
    m1 = (iota_e == i1)
    l2 = jnp.where(m1, _NEG, logits)
    v2 = jnp.max(l2, axis=0, keepdims=True)
    i2 = jnp.min(jnp.where(l2 == v2, iota_e, _NE), axis=0, keepdims=True)
    m2 = (iota_e == i2)
    a = jnp.exp(v2 - v1)
    w0 = 1.0 / (1.0 + a)

    m1f = m1.astype(jnp.float32)
    m2f = m2.astype(jnp.float32)
    ti = jax.lax.broadcasted_iota(jnp.int32, (_S, _S), 0)
    tj = jax.lax.broadcasted_iota(jnp.int32, (_S, _S), 1)
    lt = (ti < tj).astype(jnp.float32)
    c2 = jnp.concatenate([m1f, m2f], axis=0)
    p12 = jax.lax.dot_general(
        c2, lt, (((1,), (0,)), ((), ())),
        preferred_element_type=jnp.float32)
    p1 = p12[:_NE]
    p2 = p12[_NE:]
    counts1 = jnp.sum(m1f, axis=1, keepdims=True)
    counts2 = jnp.sum(m2f, axis=1, keepdims=True)
    c = counts1 + counts2
    nb = jnp.ceil(c / _BLK)
    ei = jax.lax.broadcasted_iota(jnp.int32, (_NE, _NE), 0)
    ej = jax.lax.broadcasted_iota(jnp.int32, (_NE, _NE), 1)
    lt8 = (ej < ei).astype(jnp.float32)
    snb = jax.lax.dot_general(
        lt8, nb, (((1,), (0,)), ((), ())),
        preferred_element_type=jnp.float32)
    estart = _BLK * snb

    rank0 = jnp.sum(m1f * p1, axis=0, keepdims=True)
    rank1 = jnp.sum(m2f * (p2 + counts1), axis=0, keepdims=True)
    base0 = jnp.sum(m1f * estart, axis=0, keepdims=True)
    base1 = jnp.sum(m2f * estart, axis=0, keepdims=True)
    dest_ref[0:1, :] = (base0 + rank0).astype(jnp.int32)
    dest_ref[1:2, :] = (base1 + rank1).astype(jnp.int32)
    wts_ref[0:1, :] = w0
    wts_ref[1:2, :] = 1.0 - w0

    ends = (snb + nb) * 1.0
    bio = jax.lax.broadcasted_iota(jnp.int32, (_NE, 128), 1).astype(jnp.float32)
    cnt = jnp.sum((bio >= ends).astype(jnp.float32), axis=0, keepdims=True)
    be_ref[...] = jnp.minimum(cnt, _NE - 1.0).astype(jnp.int32)
    total = jnp.sum(nb, axis=0, keepdims=True)
    bv_ref[...] = (bio[0:1, :] < total).astype(jnp.int32)


@jax.jit
def _route(x, router_w):
    return pl.pallas_call(
        _route_body,
        grid=(1,),
        in_specs=[
            pl.BlockSpec((_S, _D), lambda i: (0, 0)),
            pl.BlockSpec((_NE, _D), lambda i: (0, 0)),
        ],
        out_specs=[
            pl.BlockSpec((2, _S), lambda i: (0, 0)),
            pl.BlockSpec((2, _S), lambda i: (0, 0)),
            pl.BlockSpec((1, 128), lambda i: (0, 0)),
            pl.BlockSpec((1, 128), lambda i: (0, 0)),
        ],
        out_shape=[
            jax.ShapeDtypeStruct((2, _S), jnp.int32),
            jax.ShapeDtypeStruct((2, _S), jnp.float32),
            jax.ShapeDtypeStruct((1, 128), jnp.int32),
            jax.ShapeDtypeStruct((1, 128), jnp.int32),
        ],
    )(x, router_w)


def _ffn_body(be_ref, bv_ref, xs_ref, gw_ref, uw_ref, dw_ref, yp_ref):
    b = pl.program_id(0)

    @pl.when(bv_ref[b] == 1)
    def _():
        xb = xs_ref[...].astype(jnp.bfloat16)
        g = jax.lax.dot_general(
            xb, gw_ref[0], (((1,), (1,)), ((), ())),
            preferred_element_type=jnp.float32)
        u = jax.lax.dot_general(
            xb, uw_ref[0], (((1,), (1,)), ((), ())),
            preferred_element_type=jnp.float32)
        h = (g * jax.nn.sigmoid(g) * u).astype(jnp.bfloat16)
        yp_ref[...] = jax.lax.dot_general(
            h, dw_ref[0], (((1,), (1,)), ((), ())),
            preferred_element_type=jnp.float32)


@jax.jit
def _ffn(xs, gw, uw, dw, be, bv):
    grid_spec = pltpu.PrefetchScalarGridSpec(
        num_scalar_prefetch=2,
        grid=(_NB,),
        in_specs=[
            pl.BlockSpec((_BLK, _D), lambda b, be, bv: (b, 0)),
            pl.BlockSpec((1, _F, _D), lambda b, be, bv: (be[b], 0, 0)),
            pl.BlockSpec((1, _F, _D), lambda b, be, bv: (be[b], 0, 0)),
            pl.BlockSpec((1, _D, _F), lambda b, be, bv: (be[b], 0, 0)),
        ],
        out_specs=pl.BlockSpec((_BLK, _D), lambda b, be, bv: (b, 0)),
    )
    return pl.pallas_call(
        _ffn_body,
        grid_spec=grid_spec,
        out_shape=jax.ShapeDtypeStruct((_P, _D), jnp.float32),
        compiler_params=pltpu.CompilerParams(
            dimension_semantics=("arbitrary",),
        ),
    )(be, bv, xs, gw, uw, dw)


_MESH = plsc.VectorSubcoreMesh(core_axis_name="c", subcore_axis_name="s")


def _wid():
    return lax.axis_index("s") * 2 + lax.axis_index("c")


def _dispatch_body(x_hbm, dest_hbm, xs_hbm, idx_v, buf, sem):
    w = _wid()
    k = w // 16
    t0 = (w % 16) * 128
    for j in range(4):
        tj = t0 + j * 32
        pltpu.sync_copy(dest_hbm.at[k, pl.ds(tj, 32)], idx_v.at[j])
        pltpu.sync_copy(x_hbm.at[pl.ds(tj, 32)], buf)
        pltpu.async_copy(buf, xs_hbm.at[idx_v.at[j]], sem).wait()


@jax.jit
def _dispatch(x, dest):
    f = functools.partial(
        pl.kernel, mesh=_MESH,
        out_type=jax.ShapeDtypeStruct((_P, _D), jnp.float32),
        scratch_types=[
            pltpu.VMEM((4, 32), jnp.int32),
            pltpu.VMEM((32, _D), jnp.float32),
            pltpu.SemaphoreType.DMA,
        ],
    )(_dispatch_body)
    return f(x, dest)


def _gather2_body(yp_hbm, dest_hbm, r0_hbm, r1_hbm, idx0, idx1, b0, sem):
    w = _wid()
    t0 = w * 64
    for j in range(2):
        tj = t0 + j * 32
        pltpu.sync_copy(dest_hbm.at[0, pl.ds(tj, 32)], idx0.at[j])
        pltpu.sync_copy(dest_hbm.at[1, pl.ds(tj, 32)], idx1.at[j])
        pltpu.async_copy(yp_hbm.at[idx0.at[j]], b0, sem).wait()
        pltpu.sync_copy(b0, r0_hbm.at[pl.ds(tj, 32)])
        pltpu.async_copy(yp_hbm.at[idx1.at[j]], b0, sem).wait()
        pltpu.sync_copy(b0, r1_hbm.at[pl.ds(tj, 32)])


@jax.jit
def _gather2(yp, dest):
    f = functools.partial(
        pl.kernel, mesh=_MESH,
        out_type=[
            jax.ShapeDtypeStruct((_S, _D), jnp.float32),
            jax.ShapeDtypeStruct((_S, _D), jnp.float32),
        ],
        scratch_types=[
            pltpu.VMEM((2, 32), jnp.int32),
            pltpu.VMEM((2, 32), jnp.int32),
            pltpu.VMEM((32, _D), jnp.float32),
            pltpu.SemaphoreType.DMA,
        ],
    )(_gather2_body)
    return f(yp, dest)


def _mix_body(x_ref, rw_ref, r0_ref, r1_ref, out_ref):
    logits = jax.lax.dot_general(
        x_ref[...], rw_ref[...], (((1,), (1,)), ((), ())),
        preferred_element_type=jnp.float32)
    iota = jax.lax.broadcasted_iota(jnp.int32, logits.shape, 1)
    v1 = jnp.max(logits, axis=1, keepdims=True)
    i1 = jnp.min(jnp.where(logits == v1, iota, _NE), axis=1, keepdims=True)
    l2 = jnp.where(iota == i1, _NEG, logits)
    v2 = jnp.max(l2, axis=1, keepdims=True)
    a = jnp.exp(v2 - v1)
    w0 = 1.0 / (1.0 + a)
    out_ref[...] = w0 * r0_ref[...] + (1.0 - w0) * r1_ref[...]


@jax.jit
def _mix(x, router_w, r0, r1):
    tb = 512
    return pl.pallas_call(
        _mix_body,
        grid=(_S // tb,),
        in_specs=[
            pl.BlockSpec((tb, _D), lambda t: (t, 0)),
            pl.BlockSpec((_NE, _D), lambda t: (0, 0)),
            pl.BlockSpec((tb, _D), lambda t: (t, 0)),
            pl.BlockSpec((tb, _D), lambda t: (t, 0)),
        ],
        out_specs=pl.BlockSpec((tb, _D), lambda t: (t, 0)),
        out_shape=jax.ShapeDtypeStruct((_S, _D), jnp.float32),
    )(x, router_w, r0, r1)


def kernel(hidden_states, router_w, gate_w, up_w, down_w):
    B, S, D = hidden_states.shape
    x = hidden_states.reshape(S, D)
    dest, wts, be, bv = _route(x, router_w)
    be1 = be.reshape(128)
    bv1 = bv.reshape(128)
    xs = _dispatch(x, dest)
    yp = _ffn(xs, gate_w.astype(jnp.bfloat16), up_w.astype(jnp.bfloat16),
              down_w.astype(jnp.bfloat16), be1, bv1)
    r0, r1 = _gather2(yp, dest)
    out = _mix(x, router_w, r0, r1)
    return out.reshape(B, S, D)
